# Initial kernel scaffold; baseline (speedup 1.0000x reference)
import jax
import jax.numpy as jnp
from jax.experimental import pallas as pl


def kernel(h, W_enc, pre_bias, enc_bias):
    raise NotImplementedError("write your pallas kernel here")


# speedup vs baseline: 4.8497x; 4.8497x over previous
import functools

import jax
import jax.numpy as jnp
from jax.experimental import pallas as pl

D_MODEL = 3072
D_SPARSE = 24576
K_SPARSE = 128
BATCH = 128

_BN = 1024
_BR = 8


def _matmul_kernel(h_ref, w_ref, pb_ref, eb_ref, out_ref):
    x = h_ref[...] - pb_ref[...]
    acts = jax.lax.dot_general(
        x, w_ref[...],
        dimension_numbers=(((1,), (1,)), ((), ())),
        preferred_element_type=jnp.float32,
    )
    out_ref[...] = acts + eb_ref[...]


def _select_kernel(a_ref, out_ref):
    a = a_ref[...]
    s = jax.lax.bitcast_convert_type(a, jnp.int32)
    y = jnp.where(s >= 0, s, s ^ jnp.int32(0x7FFFFFFF))

    k = jnp.int32(K_SPARSE)

    def vbody(i, t):
        b = 31 - i
        cand = t + (jnp.int32(1) << b)
        cnt = jnp.sum((y >= cand).astype(jnp.int32), axis=1, keepdims=True)
        return jnp.where(cnt >= k, cand, t)

    t0 = jnp.full((a.shape[0], 1), jnp.int32(-2147483648))
    t = jax.lax.fori_loop(0, 32, vbody, t0)

    cnt_gt = jnp.sum((y > t).astype(jnp.int32), axis=1, keepdims=True)
    extras = k - cnt_gt
    idx = jax.lax.broadcasted_iota(jnp.int32, a.shape, 1)
    tie = y == t

    def ibody(i, m):
        b = 14 - i
        cand = m + (jnp.int32(1) << b)
        cnt = jnp.sum((tie & (idx <= cand)).astype(jnp.int32), axis=1,
                      keepdims=True)
        return jnp.where(cnt < extras, cand, m)

    m0 = jnp.full((a.shape[0], 1), jnp.int32(-1))
    m = jax.lax.fori_loop(0, 15, ibody, m0)

    mask = (y > t) | (tie & (idx <= m + 1))
    out_ref[...] = jnp.where(mask, jnp.maximum(a, 0.0), 0.0)


@jax.jit
def kernel(h, W_enc, pre_bias, enc_bias):
    pb = pre_bias.reshape(1, D_MODEL)
    eb = enc_bias.reshape(1, D_SPARSE)

    pre_acts = pl.pallas_call(
        _matmul_kernel,
        grid=(D_SPARSE // _BN,),
        in_specs=[
            pl.BlockSpec((BATCH, D_MODEL), lambda i: (0, 0)),
            pl.BlockSpec((_BN, D_MODEL), lambda i: (i, 0)),
            pl.BlockSpec((1, D_MODEL), lambda i: (0, 0)),
            pl.BlockSpec((1, _BN), lambda i: (0, i)),
        ],
        out_specs=pl.BlockSpec((BATCH, _BN), lambda i: (0, i)),
        out_shape=jax.ShapeDtypeStruct((BATCH, D_SPARSE), jnp.float32),
    )(h, W_enc, pb, eb)

    out = pl.pallas_call(
        _select_kernel,
        grid=(BATCH // _BR,),
        in_specs=[pl.BlockSpec((_BR, D_SPARSE), lambda i: (i, 0))],
        out_specs=pl.BlockSpec((_BR, D_SPARSE), lambda i: (i, 0)),
        out_shape=jax.ShapeDtypeStruct((BATCH, D_SPARSE), jnp.float32),
    )(pre_acts)
    return out


# speedup vs baseline: 9.6144x; 1.9825x over previous
import functools

import jax
import jax.numpy as jnp
from jax.experimental import pallas as pl

D_MODEL = 3072
D_SPARSE = 24576
K_SPARSE = 128
BATCH = 128

_BN = 1024
_BR = 32


def _matmul_kernel(h_ref, w_ref, pb_ref, eb_ref, out_ref):
    x = h_ref[...] - pb_ref[...]
    acts = jax.lax.dot_general(
        x, w_ref[...],
        dimension_numbers=(((1,), (1,)), ((), ())),
        preferred_element_type=jnp.float32,
    )
    out_ref[...] = acts + eb_ref[...]


def _select_kernel(a_ref, out_ref):
    a = a_ref[...]
    s = jax.lax.bitcast_convert_type(a, jnp.int32)
    y = jnp.where(s >= 0, s, s ^ jnp.int32(0x7FFFFFFF))

    k = jnp.int32(K_SPARSE)

    t = jnp.full((a.shape[0], 1), jnp.int32(-2147483648))
    for b in range(31, -1, -1):
        cand = t + (jnp.int32(1) << b)
        cnt = jnp.sum((y >= cand).astype(jnp.int32), axis=1, keepdims=True)
        t = jnp.where(cnt >= k, cand, t)

    cnt_ge = jnp.sum((y >= t).astype(jnp.int32), axis=1, keepdims=True)
    no_ties = jnp.all(cnt_ge == k)

    @pl.when(no_ties)
    def _():
        out_ref[...] = jnp.where(y >= t, jnp.maximum(a, 0.0), 0.0)

    @pl.when(jnp.logical_not(no_ties))
    def _():
        cnt_gt = jnp.sum((y > t).astype(jnp.int32), axis=1, keepdims=True)
        extras = k - cnt_gt
        idx = jax.lax.broadcasted_iota(jnp.int32, a.shape, 1)
        tie = y == t

        def ibody(i, m):
            b = 14 - i
            cand = m + (jnp.int32(1) << b)
            cnt = jnp.sum((tie & (idx <= cand)).astype(jnp.int32), axis=1,
                          keepdims=True)
            return jnp.where(cnt < extras, cand, m)

        m0 = jnp.full((a.shape[0], 1), jnp.int32(-1))
        m = jax.lax.fori_loop(0, 15, ibody, m0)

        mask = (y > t) | (tie & (idx <= m + 1))
        out_ref[...] = jnp.where(mask, jnp.maximum(a, 0.0), 0.0)


@jax.jit
def kernel(h, W_enc, pre_bias, enc_bias):
    pb = pre_bias.reshape(1, D_MODEL)
    eb = enc_bias.reshape(1, D_SPARSE)

    pre_acts = pl.pallas_call(
        _matmul_kernel,
        grid=(D_SPARSE // _BN,),
        in_specs=[
            pl.BlockSpec((BATCH, D_MODEL), lambda i: (0, 0)),
            pl.BlockSpec((_BN, D_MODEL), lambda i: (i, 0)),
            pl.BlockSpec((1, D_MODEL), lambda i: (0, 0)),
            pl.BlockSpec((1, _BN), lambda i: (0, i)),
        ],
        out_specs=pl.BlockSpec((BATCH, _BN), lambda i: (0, i)),
        out_shape=jax.ShapeDtypeStruct((BATCH, D_SPARSE), jnp.float32),
    )(h, W_enc, pb, eb)

    out = pl.pallas_call(
        _select_kernel,
        grid=(BATCH // _BR,),
        in_specs=[pl.BlockSpec((_BR, D_SPARSE), lambda i: (i, 0))],
        out_specs=pl.BlockSpec((_BR, D_SPARSE), lambda i: (i, 0)),
        out_shape=jax.ShapeDtypeStruct((BATCH, D_SPARSE), jnp.float32),
    )(pre_acts)
    return out
